# Initial kernel scaffold; baseline (speedup 1.0000x reference)
import jax
import jax.numpy as jnp
from jax.experimental import pallas as pl


def kernel(x, pos, batch, W1, b1, W2, b2, W3, b3, W4, b4, L1, bL1, L2, bL2, M1, bM1, M2, bM2):
    raise NotImplementedError("write your pallas kernel here")


# speedup vs baseline: 8.9945x; 8.9945x over previous
import functools

import jax
import jax.numpy as jnp
from jax import lax
from jax.experimental import pallas as pl
from jax.experimental.pallas import tpu as pltpu
from jax.experimental.pallas import tpu_sc as plsc

N = 8192
K = 5
NG = 8
TR = 256
CC = 512
NT = N // TR
NCH = N // CC
F = 64
FP = 128
_BIGI = 2**31 - 1


def _lk(v):
    return jnp.where(v >= 0, v, 0.01 * v)


def _pick_min(vals, idxs, cid):
    m = jnp.min(vals, axis=1, keepdims=True)
    a = jnp.min(jnp.where(vals == m, idxs, _BIGI), axis=1, keepdims=True)
    return m, a


def _set_lane(dst, lane, src):
    li = lax.broadcasted_iota(jnp.int32, dst.shape, 1)
    return jnp.where(li == lane, jnp.broadcast_to(src, dst.shape), dst)


def _knn_ab_body(cbase_ref, ncs_ref, hr_ref, br_ref, hf_ref, bc3_ref,
                 wtd_ref, wbot_ref, ba_ref, idx_ref, a_ref, b_ref, *, d):
    t = pl.program_id(0)
    hr = hr_ref[...]
    br = br_ref[...]
    hrp = jnp.concatenate([hr, jnp.ones((TR, 1), jnp.float32)], axis=1)
    inf = jnp.float32(jnp.inf)
    bv0 = jnp.full((TR, 8), inf, jnp.float32)
    bi0 = lax.broadcasted_iota(jnp.int32, (TR, 8), 1)
    cbase = cbase_ref[t]
    ncs = ncs_ref[t]

    def chunk(j, carry):
        bv, bi = carry
        jc = cbase + j
        c = pl.multiple_of(jc * CC, CC)
        hc = hf_ref[pl.ds(c, CC), :]
        sc = jnp.sum(hc * hc, axis=1, keepdims=True)
        hcp = jnp.concatenate([hc * (-2.0), sc], axis=1)
        dist = lax.dot_general(hrp, hcp, (((1,), (1,)), ((), ())),
                               preferred_element_type=jnp.float32)
        bc = bc3_ref[pl.ds(jc, 1), 0, :].reshape(1, CC)
        cid = lax.broadcasted_iota(jnp.int32, (TR, CC), 1) + c
        dist = jnp.where(br == bc, dist, inf)
        lv = jnp.full((TR, 8), inf, jnp.float32)
        li = jnp.full((TR, 8), _BIGI, jnp.int32)
        for k in range(K):
            m, a = _pick_min(dist, cid, cid)
            lv = _set_lane(lv, k, m)
            li = _set_lane(li, k, a)
            dist = jnp.where(cid == a, inf, dist)
        cv = jnp.concatenate([bv, lv], axis=1)
        ci = jnp.concatenate([bi, li], axis=1)
        nv = jnp.full((TR, 8), inf, jnp.float32)
        ni = jnp.full((TR, 8), _BIGI, jnp.int32)
        for k in range(K):
            m, a = _pick_min(cv, ci, ci)
            nv = _set_lane(nv, k, m)
            ni = _set_lane(ni, k, a)
            cv = jnp.where(ci == a, inf, cv)
        return nv, ni

    _, bi = lax.fori_loop(0, ncs, chunk, (bv0, bi0))
    idx_ref[...] = bi
    a_ref[...] = jnp.dot(hr, wtd_ref[...],
                         preferred_element_type=jnp.float32) + ba_ref[...]
    bmat = jnp.dot(hr, wbot_ref[...], preferred_element_type=jnp.float32)
    b_ref[...] = jnp.concatenate(
        [bmat, jnp.zeros((TR, FP - F), jnp.float32)], axis=1)


def _knn_ab(h, brow, bcol3, cbase, ncs, wtd, wbot, ba):
    d = h.shape[1]
    body = functools.partial(_knn_ab_body, d=d)
    return pl.pallas_call(
        body,
        grid=(NT,),
        in_specs=[
            pl.BlockSpec(memory_space=pltpu.SMEM),
            pl.BlockSpec(memory_space=pltpu.SMEM),
            pl.BlockSpec((TR, d), lambda t: (t, 0)),
            pl.BlockSpec((TR, 1), lambda t: (t, 0)),
            pl.BlockSpec(memory_space=pltpu.VMEM),
            pl.BlockSpec(memory_space=pltpu.VMEM),
            pl.BlockSpec(memory_space=pltpu.VMEM),
            pl.BlockSpec(memory_space=pltpu.VMEM),
            pl.BlockSpec(memory_space=pltpu.VMEM),
        ],
        out_specs=[
            pl.BlockSpec((TR, 8), lambda t: (t, 0)),
            pl.BlockSpec((TR, F), lambda t: (t, 0)),
            pl.BlockSpec((TR, FP), lambda t: (t, 0)),
        ],
        out_shape=[
            jax.ShapeDtypeStruct((N, 8), jnp.int32),
            jax.ShapeDtypeStruct((N, F), jnp.float32),
            jax.ShapeDtypeStruct((N, FP), jnp.float32),
        ],
    )(cbase, ncs, h, brow, h, bcol3, wtd, wbot, ba)


def _sc_gather(table, idx):
    B = idx.shape[0]
    D = table.shape[1]
    try:
        info = plsc.get_sparse_core_info()
        nw = info.num_cores * info.num_subcores
        nc = info.num_cores
    except Exception:
        nw, nc = 32, 2
    b_per_w = B // nw
    ncnk = 1
    while (b_per_w // ncnk) * D * 4 > 360 * 1024:
        ncnk *= 2
    b_c = b_per_w // ncnk
    mesh = plsc.VectorSubcoreMesh(core_axis_name="c", subcore_axis_name="s")

    @functools.partial(
        pl.kernel, mesh=mesh,
        out_type=jax.ShapeDtypeStruct((B, D), jnp.float32),
        scratch_types=[
            pltpu.VMEM((b_c,), jnp.int32),
            pltpu.VMEM((b_c, D), jnp.float32),
            pltpu.SemaphoreType.DMA,
        ],
    )
    def k(table_hbm, idx_hbm, out_hbm, idx_v, rows_v, sem):
        wid = lax.axis_index("s") * nc + lax.axis_index("c")
        for ch in range(ncnk):
            base = wid * b_per_w + ch * b_c
            pltpu.sync_copy(idx_hbm.at[pl.ds(base, b_c)], idx_v)
            pltpu.async_copy(table_hbm.at[idx_v], rows_v, sem).wait()
            pltpu.sync_copy(rows_v, out_hbm.at[pl.ds(base, b_c)])

    return k(table, idx)


def _edge_mlp_body(a_ref, g_ref, wb_ref, bb_ref, o_ref):
    a = a_ref[...]
    wb = wb_ref[...]
    bb = bb_ref[...]
    acc = jnp.zeros((TR, F), jnp.float32)
    for k in range(K):
        e = _lk(a + g_ref[k][:, :F])
        acc = acc + _lk(jnp.dot(e, wb, preferred_element_type=jnp.float32)
                        + bb)
    o_ref[...] = acc


def _edge_mlp(a, g, wb, bb):
    return pl.pallas_call(
        _edge_mlp_body,
        grid=(NT,),
        in_specs=[
            pl.BlockSpec((TR, F), lambda t: (t, 0)),
            pl.BlockSpec((K, TR, FP), lambda t: (0, t, 0)),
            pl.BlockSpec(memory_space=pltpu.VMEM),
            pl.BlockSpec(memory_space=pltpu.VMEM),
        ],
        out_specs=pl.BlockSpec((TR, F), lambda t: (t, 0)),
        out_shape=jax.ShapeDtypeStruct((N, F), jnp.float32),
    )(a, g, wb, bb)


def _final_body(glo_ref, ghi_ref, cat_ref, br_ref, l1_ref, bl1_ref, l2_ref,
                bl2_ref, m1_ref, bm1_ref, m2_ref, bm2_ref, out_ref,
                mx_ref, mn_ref, sm_ref, cnt_ref):
    t = pl.program_id(0)
    inf = jnp.float32(jnp.inf)

    @pl.when(t == 0)
    def _init():
        mx_ref[...] = jnp.full((NG, 512), -inf, jnp.float32)
        mn_ref[...] = jnp.full((NG, 512), inf, jnp.float32)
        sm_ref[...] = jnp.zeros((NG, 512), jnp.float32)
        cnt_ref[...] = jnp.zeros((NG, 128), jnp.float32)

    h1 = _lk(jnp.dot(cat_ref[...], l1_ref[...],
                     preferred_element_type=jnp.float32) + bl1_ref[...])
    o = jnp.dot(h1, l2_ref[...],
                preferred_element_type=jnp.float32) + bl2_ref[...]
    br = br_ref[...]

    def upd(g, _):
        mask = br == g
        mxg = jnp.max(jnp.where(mask, o, -inf), axis=0, keepdims=True)
        mng = jnp.min(jnp.where(mask, o, inf), axis=0, keepdims=True)
        smg = jnp.sum(jnp.where(mask, o, 0.0), axis=0, keepdims=True)
        cg = jnp.sum(mask.astype(jnp.float32), axis=0, keepdims=True)
        mx_ref[pl.ds(g, 1), :] = jnp.maximum(mx_ref[pl.ds(g, 1), :], mxg)
        mn_ref[pl.ds(g, 1), :] = jnp.minimum(mn_ref[pl.ds(g, 1), :], mng)
        sm_ref[pl.ds(g, 1), :] = sm_ref[pl.ds(g, 1), :] + smg
        cnt_ref[pl.ds(g, 1), :] = (cnt_ref[pl.ds(g, 1), :]
                                   + jnp.broadcast_to(cg, (1, 128)))
        return 0

    lax.fori_loop(glo_ref[t], ghi_ref[t] + 1, upd, 0)

    @pl.when(t == NT - 1)
    def _head():
        sm = sm_ref[...]
        cnt = jnp.maximum(cnt_ref[:, 0:1], 1.0)
        x5 = jnp.concatenate([mx_ref[...], mn_ref[...], sm, sm / cnt],
                             axis=1)
        hh = _lk(jnp.dot(x5, m1_ref[...],
                         preferred_element_type=jnp.float32) + bm1_ref[...])
        out_ref[...] = jnp.dot(hh, m2_ref[...],
                               preferred_element_type=jnp.float32) + bm2_ref[...]


def _final(glo, ghi, cat, brow, l1, bl1, l2, bl2, m1, bm1, m2, bm2, cla):
    return pl.pallas_call(
        _final_body,
        grid=(NT,),
        in_specs=[
            pl.BlockSpec(memory_space=pltpu.SMEM),
            pl.BlockSpec(memory_space=pltpu.SMEM),
            pl.BlockSpec((TR, 4 * F), lambda t: (t, 0)),
            pl.BlockSpec((TR, 1), lambda t: (t, 0)),
            pl.BlockSpec(memory_space=pltpu.VMEM),
            pl.BlockSpec(memory_space=pltpu.VMEM),
            pl.BlockSpec(memory_space=pltpu.VMEM),
            pl.BlockSpec(memory_space=pltpu.VMEM),
            pl.BlockSpec(memory_space=pltpu.VMEM),
            pl.BlockSpec(memory_space=pltpu.VMEM),
            pl.BlockSpec(memory_space=pltpu.VMEM),
            pl.BlockSpec(memory_space=pltpu.VMEM),
        ],
        out_specs=pl.BlockSpec((NG, cla), lambda t: (0, 0)),
        out_shape=jax.ShapeDtypeStruct((NG, cla), jnp.float32),
        scratch_shapes=[
            pltpu.VMEM((NG, 512), jnp.float32),
            pltpu.VMEM((NG, 512), jnp.float32),
            pltpu.VMEM((NG, 512), jnp.float32),
            pltpu.VMEM((NG, 128), jnp.float32),
        ],
    )(glo, ghi, cat, brow, l1, bl1, l2, bl2, m1, bm1, m2, bm2)


def kernel(x, pos, batch, W1, b1, W2, b2, W3, b3, W4, b4,
           L1, bL1, L2, bL2, M1, bM1, M2, bM2):
    batch = batch.astype(jnp.int32)
    xx = jnp.concatenate([x, pos], axis=1)

    starts = jnp.searchsorted(
        batch, jnp.arange(NG + 1, dtype=jnp.int32)).astype(jnp.int32)
    t0 = jnp.arange(NT, dtype=jnp.int32) * TR
    g_lo = batch[t0]
    g_hi = batch[t0 + TR - 1]
    clo = starts[g_lo]
    chi = starts[g_hi + 1]
    cbase = clo // CC
    ncs = (chi - cbase * CC + CC - 1) // CC
    brow = batch.reshape(N, 1)
    bcol3 = batch.reshape(NCH, 1, CC)

    h = xx
    feats = []
    layers = [(W1, b1, W2, b2), (W3, b3, W4, b4),
              (W3, b3, W4, b4), (W3, b3, W4, b4)]
    for Wa, ba, Wb, bb in layers:
        d = h.shape[1]
        wtd = Wa[:d] - Wa[d:]
        wbot = Wa[d:]
        idx, a, b = _knn_ab(h, brow, bcol3, cbase, ncs, wtd, wbot, ba)
        idxf = idx[:, :K].T.reshape(-1)
        g = _sc_gather(b, idxf).reshape(K, N, FP)
        h = _edge_mlp(a, g, Wb, bb)
        feats.append(h)

    cat = jnp.concatenate(feats, axis=1)
    return _final(g_lo, g_hi, cat, brow, L1, bL1, L2, bL2,
                  M1, bM1, M2, bM2, M2.shape[1])


# speedup vs baseline: 19.0895x; 2.1224x over previous
import functools

import jax
import jax.numpy as jnp
from jax import lax
from jax.experimental import pallas as pl
from jax.experimental.pallas import tpu as pltpu
from jax.experimental.pallas import tpu_sc as plsc

N = 8192
K = 5
NG = 8
TR = 256
CC = 512
NT = N // TR
NCH = N // CC
F = 64
FP = 128
_BIGI = 2**31 - 1


def _lk(v):
    return jnp.where(v >= 0, v, 0.01 * v)


def _pick_pass(vals, cidf):
    inf = jnp.float32(jnp.inf)
    m = jnp.min(vals, axis=0, keepdims=True)
    sel = jnp.where(vals == m, cidf, inf)
    a = jnp.min(sel, axis=0, keepdims=True)
    vals = jnp.where(sel == a, inf, vals)
    return m, a, vals


def _knn_ab_body(cbase_ref, ncs_ref, hr_ref, bl_ref, hf_ref, bs_ref,
                 wtd_ref, wbot_ref, ba_ref, idx_ref, a_ref, b_ref, *, d):
    t = pl.program_id(0)
    hr = hr_ref[...]
    bl = bl_ref[...]
    hrp = jnp.concatenate([hr, jnp.ones((TR, 1), jnp.float32)], axis=1)
    inf = jnp.float32(jnp.inf)
    sub8 = lax.broadcasted_iota(jnp.int32, (8, TR), 0)
    bv0 = jnp.full((8, TR), inf, jnp.float32)
    bi0 = sub8.astype(jnp.float32)
    cbase = cbase_ref[t]
    ncs = ncs_ref[t]

    def put(dst, k, row):
        return jnp.where(sub8 == k, jnp.broadcast_to(row, (8, TR)), dst)

    def chunk(j, carry):
        bv, bi = carry
        jc = cbase + j
        c = pl.multiple_of(jc * CC, CC)
        hc = hf_ref[pl.ds(c, CC), :]
        sc = jnp.sum(hc * hc, axis=1, keepdims=True)
        hcp = jnp.concatenate([hc * (-2.0), sc], axis=1)
        dist = lax.dot_general(hcp, hrp, (((1,), (1,)), ((), ())),
                               preferred_element_type=jnp.float32)
        bc = bs_ref[pl.ds(c, CC), :]
        cidf = (lax.broadcasted_iota(jnp.int32, (CC, TR), 0).astype(
            jnp.float32) + jnp.float32(c))
        dist = jnp.where(bc == bl, dist, inf)
        lv = jnp.full((8, TR), inf, jnp.float32)
        li = jnp.full((8, TR), inf, jnp.float32)
        for k in range(K):
            m, a, dist = _pick_pass(dist, cidf)
            lv = put(lv, k, m)
            li = put(li, k, a)
        cv = jnp.concatenate([bv, lv], axis=0)
        ci = jnp.concatenate([bi, li], axis=0)
        nv = jnp.full((8, TR), inf, jnp.float32)
        ni = jnp.full((8, TR), inf, jnp.float32)
        for k in range(K):
            m, a, cv = _pick_pass(cv, ci)
            nv = put(nv, k, m)
            ni = put(ni, k, a)
        return nv, ni

    _, bi = lax.fori_loop(0, ncs, chunk, (bv0, bi0))
    idx_ref[...] = bi.astype(jnp.int32)
    a_ref[...] = jnp.dot(hr, wtd_ref[...],
                         preferred_element_type=jnp.float32) + ba_ref[...]
    bmat = jnp.dot(hr, wbot_ref[...], preferred_element_type=jnp.float32)
    b_ref[...] = jnp.concatenate(
        [bmat, jnp.zeros((TR, FP - F), jnp.float32)], axis=1)


def _knn_ab(h, blane, bsub, cbase, ncs, wtd, wbot, ba):
    d = h.shape[1]
    body = functools.partial(_knn_ab_body, d=d)
    return pl.pallas_call(
        body,
        grid=(NT,),
        in_specs=[
            pl.BlockSpec(memory_space=pltpu.SMEM),
            pl.BlockSpec(memory_space=pltpu.SMEM),
            pl.BlockSpec((TR, d), lambda t: (t, 0)),
            pl.BlockSpec((1, TR), lambda t: (0, t)),
            pl.BlockSpec(memory_space=pltpu.VMEM),
            pl.BlockSpec(memory_space=pltpu.VMEM),
            pl.BlockSpec(memory_space=pltpu.VMEM),
            pl.BlockSpec(memory_space=pltpu.VMEM),
            pl.BlockSpec(memory_space=pltpu.VMEM),
        ],
        out_specs=[
            pl.BlockSpec((8, TR), lambda t: (0, t)),
            pl.BlockSpec((TR, F), lambda t: (t, 0)),
            pl.BlockSpec((TR, FP), lambda t: (t, 0)),
        ],
        out_shape=[
            jax.ShapeDtypeStruct((8, N), jnp.int32),
            jax.ShapeDtypeStruct((N, F), jnp.float32),
            jax.ShapeDtypeStruct((N, FP), jnp.float32),
        ],
    )(cbase, ncs, h, blane, h, bsub, wtd, wbot, ba)


def _sc_gather(table, idx):
    B = idx.shape[0]
    D = table.shape[1]
    try:
        info = plsc.get_sparse_core_info()
        nw = info.num_cores * info.num_subcores
        nc = info.num_cores
    except Exception:
        nw, nc = 32, 2
    b_per_w = B // nw
    ncnk = 1
    while (b_per_w // ncnk) * D * 4 > 360 * 1024:
        ncnk *= 2
    b_c = b_per_w // ncnk
    mesh = plsc.VectorSubcoreMesh(core_axis_name="c", subcore_axis_name="s")

    @functools.partial(
        pl.kernel, mesh=mesh,
        out_type=jax.ShapeDtypeStruct((B, D), jnp.float32),
        scratch_types=[
            pltpu.VMEM((b_c,), jnp.int32),
            pltpu.VMEM((b_c, D), jnp.float32),
            pltpu.SemaphoreType.DMA,
        ],
    )
    def k(table_hbm, idx_hbm, out_hbm, idx_v, rows_v, sem):
        wid = lax.axis_index("s") * nc + lax.axis_index("c")
        for ch in range(ncnk):
            base = wid * b_per_w + ch * b_c
            pltpu.sync_copy(idx_hbm.at[pl.ds(base, b_c)], idx_v)
            pltpu.async_copy(table_hbm.at[idx_v], rows_v, sem).wait()
            pltpu.sync_copy(rows_v, out_hbm.at[pl.ds(base, b_c)])

    return k(table, idx)


def _edge_mlp_body(a_ref, g_ref, wb_ref, bb_ref, o_ref):
    a = a_ref[...]
    wb = wb_ref[...]
    bb = bb_ref[...]
    acc = jnp.zeros((TR, F), jnp.float32)
    for k in range(K):
        e = _lk(a + g_ref[k][:, :F])
        acc = acc + _lk(jnp.dot(e, wb, preferred_element_type=jnp.float32)
                        + bb)
    o_ref[...] = acc


def _edge_mlp(a, g, wb, bb):
    return pl.pallas_call(
        _edge_mlp_body,
        grid=(NT,),
        in_specs=[
            pl.BlockSpec((TR, F), lambda t: (t, 0)),
            pl.BlockSpec((K, TR, FP), lambda t: (0, t, 0)),
            pl.BlockSpec(memory_space=pltpu.VMEM),
            pl.BlockSpec(memory_space=pltpu.VMEM),
        ],
        out_specs=pl.BlockSpec((TR, F), lambda t: (t, 0)),
        out_shape=jax.ShapeDtypeStruct((N, F), jnp.float32),
    )(a, g, wb, bb)


def _final_body(glo_ref, ghi_ref, cat_ref, br_ref, l1_ref, bl1_ref, l2_ref,
                bl2_ref, m1_ref, bm1_ref, m2_ref, bm2_ref, out_ref,
                mx_ref, mn_ref, sm_ref, cnt_ref):
    t = pl.program_id(0)
    inf = jnp.float32(jnp.inf)

    @pl.when(t == 0)
    def _init():
        mx_ref[...] = jnp.full((NG, 512), -inf, jnp.float32)
        mn_ref[...] = jnp.full((NG, 512), inf, jnp.float32)
        sm_ref[...] = jnp.zeros((NG, 512), jnp.float32)
        cnt_ref[...] = jnp.zeros((NG, 128), jnp.float32)

    h1 = _lk(jnp.dot(cat_ref[...], l1_ref[...],
                     preferred_element_type=jnp.float32) + bl1_ref[...])
    o = jnp.dot(h1, l2_ref[...],
                preferred_element_type=jnp.float32) + bl2_ref[...]
    br = br_ref[...]

    def upd(g, _):
        mask = br == g
        mxg = jnp.max(jnp.where(mask, o, -inf), axis=0, keepdims=True)
        mng = jnp.min(jnp.where(mask, o, inf), axis=0, keepdims=True)
        smg = jnp.sum(jnp.where(mask, o, 0.0), axis=0, keepdims=True)
        cg = jnp.sum(mask.astype(jnp.float32), axis=0, keepdims=True)
        mx_ref[pl.ds(g, 1), :] = jnp.maximum(mx_ref[pl.ds(g, 1), :], mxg)
        mn_ref[pl.ds(g, 1), :] = jnp.minimum(mn_ref[pl.ds(g, 1), :], mng)
        sm_ref[pl.ds(g, 1), :] = sm_ref[pl.ds(g, 1), :] + smg
        cnt_ref[pl.ds(g, 1), :] = (cnt_ref[pl.ds(g, 1), :]
                                   + jnp.broadcast_to(cg, (1, 128)))
        return 0

    lax.fori_loop(glo_ref[t], ghi_ref[t] + 1, upd, 0)

    @pl.when(t == NT - 1)
    def _head():
        sm = sm_ref[...]
        cnt = jnp.maximum(cnt_ref[:, 0:1], 1.0)
        x5 = jnp.concatenate([mx_ref[...], mn_ref[...], sm, sm / cnt],
                             axis=1)
        hh = _lk(jnp.dot(x5, m1_ref[...],
                         preferred_element_type=jnp.float32) + bm1_ref[...])
        out_ref[...] = jnp.dot(hh, m2_ref[...],
                               preferred_element_type=jnp.float32) + bm2_ref[...]


def _final(glo, ghi, cat, brow, l1, bl1, l2, bl2, m1, bm1, m2, bm2, cla):
    return pl.pallas_call(
        _final_body,
        grid=(NT,),
        in_specs=[
            pl.BlockSpec(memory_space=pltpu.SMEM),
            pl.BlockSpec(memory_space=pltpu.SMEM),
            pl.BlockSpec((TR, 4 * F), lambda t: (t, 0)),
            pl.BlockSpec((TR, 1), lambda t: (t, 0)),
            pl.BlockSpec(memory_space=pltpu.VMEM),
            pl.BlockSpec(memory_space=pltpu.VMEM),
            pl.BlockSpec(memory_space=pltpu.VMEM),
            pl.BlockSpec(memory_space=pltpu.VMEM),
            pl.BlockSpec(memory_space=pltpu.VMEM),
            pl.BlockSpec(memory_space=pltpu.VMEM),
            pl.BlockSpec(memory_space=pltpu.VMEM),
            pl.BlockSpec(memory_space=pltpu.VMEM),
        ],
        out_specs=pl.BlockSpec((NG, cla), lambda t: (0, 0)),
        out_shape=jax.ShapeDtypeStruct((NG, cla), jnp.float32),
        scratch_shapes=[
            pltpu.VMEM((NG, 512), jnp.float32),
            pltpu.VMEM((NG, 512), jnp.float32),
            pltpu.VMEM((NG, 512), jnp.float32),
            pltpu.VMEM((NG, 128), jnp.float32),
        ],
    )(glo, ghi, cat, brow, l1, bl1, l2, bl2, m1, bm1, m2, bm2)


def kernel(x, pos, batch, W1, b1, W2, b2, W3, b3, W4, b4,
           L1, bL1, L2, bL2, M1, bM1, M2, bM2):
    batch = batch.astype(jnp.int32)
    xx = jnp.concatenate([x, pos], axis=1)

    starts = jnp.searchsorted(
        batch, jnp.arange(NG + 1, dtype=jnp.int32)).astype(jnp.int32)
    t0 = jnp.arange(NT, dtype=jnp.int32) * TR
    g_lo = batch[t0]
    g_hi = batch[t0 + TR - 1]
    clo = starts[g_lo]
    chi = starts[g_hi + 1]
    cbase = clo // CC
    ncs = (chi - cbase * CC + CC - 1) // CC
    brow = batch.reshape(N, 1)
    blane = batch.reshape(1, N)

    h = xx
    feats = []
    layers = [(W1, b1, W2, b2), (W3, b3, W4, b4),
              (W3, b3, W4, b4), (W3, b3, W4, b4)]
    for Wa, ba, Wb, bb in layers:
        d = h.shape[1]
        wtd = Wa[:d] - Wa[d:]
        wbot = Wa[d:]
        idx, a, b = _knn_ab(h, blane, brow, cbase, ncs, wtd, wbot, ba)
        idxf = idx[:K].reshape(-1)
        g = _sc_gather(b, idxf).reshape(K, N, FP)
        h = _edge_mlp(a, g, Wb, bb)
        feats.append(h)

    cat = jnp.concatenate(feats, axis=1)
    return _final(g_lo, g_hi, cat, brow, L1, bL1, L2, bL2,
                  M1, bM1, M2, bM2, M2.shape[1])
